# Initial kernel scaffold; baseline (speedup 1.0000x reference)
import jax
import jax.numpy as jnp
from jax.experimental import pallas as pl


def kernel(x_num, x_cat, E, W1, b1, W2, b2, W3, b3):
    raise NotImplementedError("write your pallas kernel here")


# speedup vs baseline: 4.2368x; 4.2368x over previous
import functools

import jax
import jax.numpy as jnp
from jax import lax
from jax.experimental import pallas as pl
from jax.experimental.pallas import tpu as pltpu
from jax.experimental.pallas import tpu_sc as plsc

N_FIELDS = 26
EMB_DIM = 50
EMB_PAD = 56


def _sc_gather(table, flat_idx):
    n = flat_idx.shape[0]
    info = plsc.get_sparse_core_info()
    nw = info.num_cores * info.num_subcores
    per_w = n // nw
    ch = 1024
    n_ch = per_w // ch
    mesh = plsc.VectorSubcoreMesh(core_axis_name="c", subcore_axis_name="s")

    @functools.partial(
        pl.kernel,
        mesh=mesh,
        out_type=jax.ShapeDtypeStruct((n, EMB_PAD), jnp.float32),
        scratch_types=[
            pltpu.VMEM((ch // 128, 128), jnp.int32),
            pltpu.VMEM((ch, EMB_PAD), jnp.float32),
            pltpu.SemaphoreType.DMA,
        ],
        compiler_params=pltpu.CompilerParams(use_tc_tiling_on_sc=False),
    )
    def k(table_hbm, idx2_hbm, out_hbm, idx_v, rows_v, sem):
        wid = lax.axis_index("s") * info.num_cores + lax.axis_index("c")
        base = wid * per_w

        def body(i, carry):
            off = base + i * ch
            pltpu.sync_copy(
                idx2_hbm.at[pl.ds(off // 128, ch // 128)], idx_v)
            copies = [
                pltpu.async_copy(
                    table_hbm.at[idx_v.at[j]],
                    rows_v.at[pl.ds(j * 128, 128)], sem)
                for j in range(ch // 128)
            ]
            for c in copies:
                c.wait()
            pltpu.sync_copy(rows_v, out_hbm.at[pl.ds(off, ch)])
            return carry

        lax.fori_loop(0, n_ch, body, 0)

    return k(table, flat_idx.reshape(n // 128, 128))


def _tc_mlp(x_num, emb_flat, w1n, w1e, b1, w2, b2, w3, b3):
    b = x_num.shape[0]
    d_num = x_num.shape[1]
    d_emb = emb_flat.shape[1]
    blk = 1024
    grid = (b // blk,)

    def body(xn_ref, e_ref, w1n_ref, w1e_ref, b1_ref, w2_ref, b2_ref,
             w3_ref, b3_ref, out_ref):
        h = jnp.dot(e_ref[...], w1e_ref[...], preferred_element_type=jnp.float32)
        h = h + jnp.dot(xn_ref[...], w1n_ref[...], preferred_element_type=jnp.float32)
        h = jnp.maximum(h + b1_ref[...], 0.0)
        h = jnp.maximum(
            jnp.dot(h, w2_ref[...], preferred_element_type=jnp.float32) + b2_ref[...],
            0.0)
        out_ref[...] = (
            jnp.dot(h, w3_ref[...], preferred_element_type=jnp.float32) + b3_ref[...])

    out = pl.pallas_call(
        body,
        grid=grid,
        in_specs=[
            pl.BlockSpec((blk, d_num), lambda i: (i, 0)),
            pl.BlockSpec((blk, d_emb), lambda i: (i, 0)),
            pl.BlockSpec((d_num, 128), lambda i: (0, 0)),
            pl.BlockSpec((d_emb, 128), lambda i: (0, 0)),
            pl.BlockSpec((1, 128), lambda i: (0, 0)),
            pl.BlockSpec((128, 64), lambda i: (0, 0)),
            pl.BlockSpec((1, 64), lambda i: (0, 0)),
            pl.BlockSpec((64, 1), lambda i: (0, 0)),
            pl.BlockSpec((1, 1), lambda i: (0, 0)),
        ],
        out_specs=pl.BlockSpec((blk, 1), lambda i: (i, 0)),
        out_shape=jax.ShapeDtypeStruct((b, 1), jnp.float32),
    )(x_num, emb_flat, w1n, w1e, b1.reshape(1, -1), w2, b2.reshape(1, -1),
      w3, b3.reshape(1, 1))
    return out[:, 0]


def kernel(x_num, x_cat, E, W1, b1, W2, b2, W3, b3):
    b = x_num.shape[0]
    card = E.shape[1]
    d_num = x_num.shape[1]
    table = jnp.pad(E.reshape(N_FIELDS * card, EMB_DIM),
                    ((0, 0), (0, EMB_PAD - EMB_DIM)))
    offs = (jnp.arange(N_FIELDS, dtype=jnp.int32) * card)[None, :]
    flat_idx = (x_cat + offs).reshape(-1)
    emb = _sc_gather(table, flat_idx)
    emb_flat = emb.reshape(b, N_FIELDS * EMB_PAD)
    w1e = jnp.pad(W1[d_num:].reshape(N_FIELDS, EMB_DIM, 128),
                  ((0, 0), (0, EMB_PAD - EMB_DIM), (0, 0)))
    w1e = w1e.reshape(N_FIELDS * EMB_PAD, 128)
    return _tc_mlp(x_num, emb_flat, W1[:d_num], w1e, b1, W2, b2, W3, b3)


# speedup vs baseline: 4.8523x; 1.1453x over previous
import functools

import jax
import jax.numpy as jnp
from jax import lax
from jax.experimental import pallas as pl
from jax.experimental.pallas import tpu as pltpu
from jax.experimental.pallas import tpu_sc as plsc

N_FIELDS = 26
EMB_DIM = 50
EMB_PAD = 64
N_TILES = N_FIELDS // 2


def _sc_gather_scatter(table, src_idx, dst_idx):
    n = src_idx.shape[0] * src_idx.shape[1]
    info = plsc.get_sparse_core_info()
    nw = info.num_cores * info.num_subcores
    per_w = n // nw
    ch = 1024
    n_ch = per_w // ch
    mesh = plsc.VectorSubcoreMesh(core_axis_name="c", subcore_axis_name="s")

    @functools.partial(
        pl.kernel,
        mesh=mesh,
        out_type=jax.ShapeDtypeStruct((n, EMB_PAD), jnp.float32),
        scratch_types=[
            pltpu.VMEM((ch // 128, 128), jnp.int32),
            pltpu.VMEM((ch // 128, 128), jnp.int32),
            pltpu.VMEM((ch, EMB_PAD), jnp.float32),
            pltpu.SemaphoreType.DMA,
            pltpu.SemaphoreType.DMA,
        ],
        compiler_params=pltpu.CompilerParams(use_tc_tiling_on_sc=False),
    )
    def k(table_hbm, src_hbm, dst_hbm, out_hbm, src_v, dst_v, rows_v,
          gsem, ssem):
        wid = lax.axis_index("s") * info.num_cores + lax.axis_index("c")
        base = wid * per_w

        def body(i, carry):
            off = base + i * ch
            pltpu.sync_copy(src_hbm.at[pl.ds(off // 128, ch // 128)], src_v)
            pltpu.sync_copy(dst_hbm.at[pl.ds(off // 128, ch // 128)], dst_v)
            gathers = [
                pltpu.async_copy(
                    table_hbm.at[src_v.at[j]],
                    rows_v.at[pl.ds(j * 128, 128)], gsem)
                for j in range(ch // 128)
            ]
            for c in gathers:
                c.wait()
            scatters = [
                pltpu.async_copy(
                    rows_v.at[pl.ds(j * 128, 128)],
                    out_hbm.at[dst_v.at[j]], ssem)
                for j in range(ch // 128)
            ]
            for c in scatters:
                c.wait()
            return carry

        lax.fori_loop(0, n_ch, body, 0)

    return k(table, src_idx, dst_idx)


def _tc_mlp(x_num, emb4, w1n, w1e, b1, w2, b2, w3, b3):
    b = x_num.shape[0]
    d_num = x_num.shape[1]
    blk = 1024
    grid = (b // blk,)

    def body(xn_ref, e_ref, w1n_ref, w1e_ref, b1_ref, w2_ref, b2_ref,
             w3_ref, b3_ref, out_ref):
        h = jnp.dot(xn_ref[...], w1n_ref[...],
                    preferred_element_type=jnp.float32)
        for t in range(N_TILES):
            xt = e_ref[:, t, :, :].reshape(blk, 128)
            h += jnp.dot(xt, w1e_ref[t],
                         preferred_element_type=jnp.float32)
        h = jnp.maximum(h + b1_ref[...], 0.0)
        h = jnp.maximum(
            jnp.dot(h, w2_ref[...], preferred_element_type=jnp.float32)
            + b2_ref[...], 0.0)
        out_ref[...] = (
            jnp.dot(h, w3_ref[...], preferred_element_type=jnp.float32)
            + b3_ref[...])

    out = pl.pallas_call(
        body,
        grid=grid,
        in_specs=[
            pl.BlockSpec((blk, d_num), lambda i: (i, 0)),
            pl.BlockSpec((blk // 8, N_TILES, 8, 128), lambda i: (i, 0, 0, 0)),
            pl.BlockSpec((d_num, 128), lambda i: (0, 0)),
            pl.BlockSpec((N_TILES, 128, 128), lambda i: (0, 0, 0)),
            pl.BlockSpec((1, 128), lambda i: (0, 0)),
            pl.BlockSpec((128, 64), lambda i: (0, 0)),
            pl.BlockSpec((1, 64), lambda i: (0, 0)),
            pl.BlockSpec((64, 1), lambda i: (0, 0)),
            pl.BlockSpec((1, 1), lambda i: (0, 0)),
        ],
        out_specs=pl.BlockSpec((blk, 1), lambda i: (i, 0)),
        out_shape=jax.ShapeDtypeStruct((b, 1), jnp.float32),
    )(x_num, emb4, w1n, w1e, b1.reshape(1, -1), w2, b2.reshape(1, -1),
      w3, b3.reshape(1, 1))
    return out[:, 0]


def kernel(x_num, x_cat, E, W1, b1, W2, b2, W3, b3):
    b = x_num.shape[0]
    card = E.shape[1]
    d_num = x_num.shape[1]
    table = jnp.pad(E, ((0, 0), (0, 0), (0, EMB_PAD - EMB_DIM)))
    table = table.reshape(N_FIELDS * card, EMB_PAD)

    offs = (jnp.arange(N_FIELDS, dtype=jnp.int32) * card)[None, :]
    src_idx = (x_cat + offs).reshape(b * N_FIELDS // 128, 128)
    bb = jnp.arange(b, dtype=jnp.int32)[:, None]
    ff = jnp.arange(N_FIELDS, dtype=jnp.int32)[None, :]
    dst = ((bb // 8) * (N_TILES * 16) + (ff // 2) * 16 + (bb % 8) * 2
           + (ff % 2))
    dst_idx = dst.reshape(b * N_FIELDS // 128, 128)

    emb = _sc_gather_scatter(table, src_idx, dst_idx)
    emb4 = emb.reshape(b // 8, N_TILES, 8, 128)

    w1e = jnp.pad(W1[d_num:].reshape(N_FIELDS, EMB_DIM, 128),
                  ((0, 0), (0, EMB_PAD - EMB_DIM), (0, 0)))
    w1e = w1e.reshape(N_TILES, 128, 128)
    return _tc_mlp(x_num, emb4, W1[:d_num], w1e, b1, W2, b2, W3, b3)


# speedup vs baseline: 6.3628x; 1.3113x over previous
import functools

import jax
import jax.numpy as jnp
from jax import lax
from jax.experimental import pallas as pl
from jax.experimental.pallas import tpu as pltpu
from jax.experimental.pallas import tpu_sc as plsc

N_FIELDS = 26
EMB_DIM = 50
LANE = 128
VB = 1024


def _tc_transpose_table(Et, card):
    grid = (N_FIELDS, pl.cdiv(card, VB))

    def body(in_ref, out_ref):
        x = in_ref[0]
        xt = jnp.swapaxes(x, 0, 1)
        out_ref[0] = jnp.pad(xt, ((0, 0), (0, LANE - EMB_DIM)))

    return pl.pallas_call(
        body,
        grid=grid,
        in_specs=[pl.BlockSpec((1, EMB_DIM, VB), lambda f, v: (f, 0, v))],
        out_specs=pl.BlockSpec((1, VB, LANE), lambda f, v: (f, v, 0)),
        out_shape=jax.ShapeDtypeStruct((N_FIELDS, card, LANE), jnp.float32),
    )(Et)


def _sc_gather_scatter(table, src_idx, dst_idx):
    n = src_idx.shape[0] * src_idx.shape[1]
    info = plsc.get_sparse_core_info()
    nw = info.num_cores * info.num_subcores
    per_w = n // nw
    ch = 512
    pair = 1024
    n_ch = per_w // pair
    mesh = plsc.VectorSubcoreMesh(core_axis_name="c", subcore_axis_name="s")

    @functools.partial(
        pl.kernel,
        mesh=mesh,
        out_type=jax.ShapeDtypeStruct((n, LANE), jnp.float32),
        scratch_types=[
            pltpu.VMEM((8, 128), jnp.int32),
            pltpu.VMEM((8, 128), jnp.int32),
            pltpu.VMEM((ch, LANE), jnp.float32),
            pltpu.SemaphoreType.DMA,
            pltpu.SemaphoreType.DMA,
        ],
    )
    def k(table_hbm, src_hbm, dst_hbm, out_hbm, src_v, dst_v, rows_v,
          gsem, ssem):
        wid = lax.axis_index("s") * info.num_cores + lax.axis_index("c")
        base = wid * per_w

        def body(i, carry):
            off = base + i * pair
            row0 = pl.multiple_of(off // 128, 8)
            pltpu.sync_copy(src_hbm.at[pl.ds(row0, 8)], src_v)
            pltpu.sync_copy(dst_hbm.at[pl.ds(row0, 8)], dst_v)
            for half in range(2):
                gathers = [
                    pltpu.async_copy(
                        table_hbm.at[src_v.at[half * 4 + j]],
                        rows_v.at[pl.ds(j * 128, 128)], gsem)
                    for j in range(ch // 128)
                ]
                for c in gathers:
                    c.wait()
                scatters = [
                    pltpu.async_copy(
                        rows_v.at[pl.ds(j * 128, 128)],
                        out_hbm.at[dst_v.at[half * 4 + j]], ssem)
                    for j in range(ch // 128)
                ]
                for c in scatters:
                    c.wait()
            return carry

        lax.fori_loop(0, n_ch, body, 0)

    return k(table, src_idx, dst_idx)


def _tc_mlp(x_num, emb4, w1n, w1e, b1, w2, b2, w3, b3):
    b = x_num.shape[0]
    d_num = x_num.shape[1]
    blk = 1024
    grid = (b // blk,)

    def body(xn_ref, e_ref, w1n_ref, w1e_ref, b1_ref, w2_ref, b2_ref,
             w3_ref, b3_ref, out_ref):
        h = jnp.dot(xn_ref[...], w1n_ref[...],
                    preferred_element_type=jnp.float32)
        for t in range(N_FIELDS):
            xt = e_ref[:, t, :, :].reshape(blk, LANE)
            h += jnp.dot(xt, w1e_ref[t],
                         preferred_element_type=jnp.float32)
        h = jnp.maximum(h + b1_ref[...], 0.0)
        h = jnp.maximum(
            jnp.dot(h, w2_ref[...], preferred_element_type=jnp.float32)
            + b2_ref[...], 0.0)
        out_ref[...] = (
            jnp.dot(h, w3_ref[...], preferred_element_type=jnp.float32)
            + b3_ref[...])

    out = pl.pallas_call(
        body,
        grid=grid,
        in_specs=[
            pl.BlockSpec((blk, d_num), lambda i: (i, 0)),
            pl.BlockSpec((blk // 8, N_FIELDS, 8, LANE),
                         lambda i: (i, 0, 0, 0)),
            pl.BlockSpec((d_num, 128), lambda i: (0, 0)),
            pl.BlockSpec((N_FIELDS, LANE, 128), lambda i: (0, 0, 0)),
            pl.BlockSpec((1, 128), lambda i: (0, 0)),
            pl.BlockSpec((128, 64), lambda i: (0, 0)),
            pl.BlockSpec((1, 64), lambda i: (0, 0)),
            pl.BlockSpec((64, 1), lambda i: (0, 0)),
            pl.BlockSpec((1, 1), lambda i: (0, 0)),
        ],
        out_specs=pl.BlockSpec((blk, 1), lambda i: (i, 0)),
        out_shape=jax.ShapeDtypeStruct((b, 1), jnp.float32),
    )(x_num, emb4, w1n, w1e, b1.reshape(1, -1), w2, b2.reshape(1, -1),
      w3, b3.reshape(1, 1))
    return out[:, 0]


def kernel(x_num, x_cat, E, W1, b1, W2, b2, W3, b3):
    b = x_num.shape[0]
    card = E.shape[1]
    d_num = x_num.shape[1]

    Et = jnp.swapaxes(E, 1, 2)
    table = _tc_transpose_table(Et, card).reshape(N_FIELDS * card, LANE)

    offs = (jnp.arange(N_FIELDS, dtype=jnp.int32) * card)[None, :]
    src_idx = (x_cat + offs).reshape(b * N_FIELDS // 128, 128)
    bb = jnp.arange(b, dtype=jnp.int32)[:, None]
    ff = jnp.arange(N_FIELDS, dtype=jnp.int32)[None, :]
    dst = ((bb // 8) * N_FIELDS + ff) * 8 + (bb % 8)
    dst_idx = dst.reshape(b * N_FIELDS // 128, 128)

    emb = _sc_gather_scatter(table, src_idx, dst_idx)
    emb4 = emb.reshape(b // 8, N_FIELDS, 8, LANE)

    w1e = jnp.pad(W1[d_num:].reshape(N_FIELDS, EMB_DIM, 128),
                  ((0, 0), (0, LANE - EMB_DIM), (0, 0)))
    return _tc_mlp(x_num, emb4, W1[:d_num], w1e, b1, W2, b2, W3, b3)


# speedup vs baseline: 11.4893x; 1.8057x over previous
import functools

import jax
import jax.numpy as jnp
from jax import lax
from jax.experimental import pallas as pl
from jax.experimental.pallas import tpu as pltpu
from jax.experimental.pallas import tpu_sc as plsc

N_FIELDS = 26
EMB_DIM = 50
LANE = 128
VB = 4096


def _tc_transpose_table(Et, card):
    grid = (N_FIELDS, pl.cdiv(card, VB))

    def body(in_ref, out_ref):
        x = in_ref[0]
        xt = jnp.swapaxes(x, 0, 1)
        out_ref[0] = jnp.pad(xt, ((0, 0), (0, LANE - EMB_DIM)))

    return pl.pallas_call(
        body,
        grid=grid,
        in_specs=[pl.BlockSpec((1, EMB_DIM, VB), lambda f, v: (f, 0, v))],
        out_specs=pl.BlockSpec((1, VB, LANE), lambda f, v: (f, v, 0)),
        out_shape=jax.ShapeDtypeStruct((N_FIELDS, card, LANE), jnp.float32),
    )(Et)


def _sc_gather_scatter(table, src_idx, dst_idx):
    n = src_idx.shape[0] * src_idx.shape[1]
    info = plsc.get_sparse_core_info()
    nw = info.num_cores * info.num_subcores
    per_w = n // nw
    ch = 512
    pair = 1024
    n_ch = per_w // pair
    mesh = plsc.VectorSubcoreMesh(core_axis_name="c", subcore_axis_name="s")

    @functools.partial(
        pl.kernel,
        mesh=mesh,
        out_type=jax.ShapeDtypeStruct((n, LANE), jnp.float32),
        scratch_types=[
            pltpu.VMEM((8, 128), jnp.int32),
            pltpu.VMEM((8, 128), jnp.int32),
            pltpu.VMEM((ch, LANE), jnp.float32),
            pltpu.SemaphoreType.DMA,
            pltpu.SemaphoreType.DMA,
        ],
    )
    def k(table_hbm, src_hbm, dst_hbm, out_hbm, src_v, dst_v, rows_v,
          gsem, ssem):
        wid = lax.axis_index("s") * info.num_cores + lax.axis_index("c")
        base = wid * per_w

        def body(i, carry):
            off = base + i * pair
            row0 = pl.multiple_of(off // 128, 8)
            pltpu.sync_copy(src_hbm.at[pl.ds(row0, 8)], src_v)
            pltpu.sync_copy(dst_hbm.at[pl.ds(row0, 8)], dst_v)
            for half in range(2):
                gathers = [
                    pltpu.async_copy(
                        table_hbm.at[src_v.at[half * 4 + j]],
                        rows_v.at[pl.ds(j * 128, 128)], gsem)
                    for j in range(ch // 128)
                ]
                for c in gathers:
                    c.wait()
                scatters = [
                    pltpu.async_copy(
                        rows_v.at[pl.ds(j * 128, 128)],
                        out_hbm.at[dst_v.at[half * 4 + j]], ssem)
                    for j in range(ch // 128)
                ]
                for c in scatters:
                    c.wait()
            return carry

        lax.fori_loop(0, n_ch, body, 0)

    return k(table, src_idx, dst_idx)


def _tc_mlp(x_num, emb4, w1n, w1e, b1, w2, b2, w3, b3):
    b = x_num.shape[0]
    d_num = x_num.shape[1]
    blk = 1024
    grid = (b // blk,)

    def body(xn_ref, e_ref, w1n_ref, w1e_ref, b1_ref, w2_ref, b2_ref,
             w3_ref, b3_ref, out_ref):
        h = jnp.dot(xn_ref[...], w1n_ref[...],
                    preferred_element_type=jnp.float32)
        for t in range(N_FIELDS):
            xt = e_ref[:, t, :, :].reshape(blk, LANE)
            h += jnp.dot(xt, w1e_ref[t],
                         preferred_element_type=jnp.float32)
        h = jnp.maximum(h + b1_ref[...], 0.0)
        h = jnp.maximum(
            jnp.dot(h, w2_ref[...], preferred_element_type=jnp.float32)
            + b2_ref[...], 0.0)
        out_ref[...] = (
            jnp.dot(h, w3_ref[...], preferred_element_type=jnp.float32)
            + b3_ref[...])

    out = pl.pallas_call(
        body,
        grid=grid,
        in_specs=[
            pl.BlockSpec((blk, d_num), lambda i: (i, 0)),
            pl.BlockSpec((blk // 8, N_FIELDS, 8, LANE),
                         lambda i: (i, 0, 0, 0)),
            pl.BlockSpec((d_num, 128), lambda i: (0, 0)),
            pl.BlockSpec((N_FIELDS, LANE, 128), lambda i: (0, 0, 0)),
            pl.BlockSpec((1, 128), lambda i: (0, 0)),
            pl.BlockSpec((128, 64), lambda i: (0, 0)),
            pl.BlockSpec((1, 64), lambda i: (0, 0)),
            pl.BlockSpec((64, 1), lambda i: (0, 0)),
            pl.BlockSpec((1, 1), lambda i: (0, 0)),
        ],
        out_specs=pl.BlockSpec((blk, 1), lambda i: (i, 0)),
        out_shape=jax.ShapeDtypeStruct((b, 1), jnp.float32),
    )(x_num, emb4, w1n, w1e, b1.reshape(1, -1), w2, b2.reshape(1, -1),
      w3, b3.reshape(1, 1))
    return out[:, 0]


def kernel(x_num, x_cat, E, W1, b1, W2, b2, W3, b3):
    b = x_num.shape[0]
    card = E.shape[1]
    d_num = x_num.shape[1]

    Et = jnp.swapaxes(E, 1, 2)
    table = _tc_transpose_table(Et, card).reshape(N_FIELDS * card, LANE)

    offs = (jnp.arange(N_FIELDS, dtype=jnp.int32) * card)[None, :]
    src_idx = (x_cat + offs).reshape(b * N_FIELDS // 128, 128)
    bb = jnp.arange(b, dtype=jnp.int32)[:, None]
    ff = jnp.arange(N_FIELDS, dtype=jnp.int32)[None, :]
    dst = ((bb // 8) * N_FIELDS + ff) * 8 + (bb % 8)
    dst_idx = dst.reshape(b * N_FIELDS // 128, 128)

    emb = _sc_gather_scatter(table, src_idx, dst_idx)
    emb4 = emb.reshape(b // 8, N_FIELDS, 8, LANE)

    w1e = jnp.pad(W1[d_num:].reshape(N_FIELDS, EMB_DIM, 128),
                  ((0, 0), (0, LANE - EMB_DIM), (0, 0)))
    return _tc_mlp(x_num, emb4, W1[:d_num], w1e, b1, W2, b2, W3, b3)


# speedup vs baseline: 13.4926x; 1.1744x over previous
import functools

import jax
import jax.numpy as jnp
from jax import lax
from jax.experimental import pallas as pl
from jax.experimental.pallas import tpu as pltpu
from jax.experimental.pallas import tpu_sc as plsc

N_FIELDS = 26
EMB_DIM = 50
LANE = 128
VB = 8192


def _tc_transpose_table(Et, card):
    grid = (N_FIELDS, pl.cdiv(card, VB))

    def body(in_ref, out_ref):
        x = in_ref[0]
        xt = jnp.swapaxes(x, 0, 1)
        out_ref[0] = jnp.pad(xt, ((0, 0), (0, LANE - EMB_DIM)))

    return pl.pallas_call(
        body,
        grid=grid,
        in_specs=[pl.BlockSpec((1, EMB_DIM, VB), lambda f, v: (f, 0, v))],
        out_specs=pl.BlockSpec((1, VB, LANE), lambda f, v: (f, v, 0)),
        out_shape=jax.ShapeDtypeStruct((N_FIELDS, card, LANE), jnp.float32),
    )(Et)


def _sc_gather_scatter(table, src_idx, dst_idx):
    n = src_idx.shape[0] * src_idx.shape[1]
    info = plsc.get_sparse_core_info()
    nw = info.num_cores * info.num_subcores
    per_w = n // nw
    ch = 512
    pair = 1024
    n_ch = per_w // pair
    mesh = plsc.VectorSubcoreMesh(core_axis_name="c", subcore_axis_name="s")

    @functools.partial(
        pl.kernel,
        mesh=mesh,
        out_type=jax.ShapeDtypeStruct((n, LANE), jnp.float32),
        scratch_types=[
            pltpu.VMEM((8, 128), jnp.int32),
            pltpu.VMEM((8, 128), jnp.int32),
            pltpu.VMEM((ch, LANE), jnp.float32),
            pltpu.SemaphoreType.DMA,
            pltpu.SemaphoreType.DMA,
        ],
    )
    def k(table_hbm, src_hbm, dst_hbm, out_hbm, src_v, dst_v, rows_v,
          gsem, ssem):
        wid = lax.axis_index("s") * info.num_cores + lax.axis_index("c")
        base = wid * per_w

        def body(i, carry):
            off = base + i * pair
            row0 = pl.multiple_of(off // 128, 8)
            pltpu.sync_copy(src_hbm.at[pl.ds(row0, 8)], src_v)
            pltpu.sync_copy(dst_hbm.at[pl.ds(row0, 8)], dst_v)
            for half in range(2):
                gathers = [
                    pltpu.async_copy(
                        table_hbm.at[src_v.at[half * 4 + j]],
                        rows_v.at[pl.ds(j * 128, 128)], gsem)
                    for j in range(ch // 128)
                ]
                for c in gathers:
                    c.wait()
                scatters = [
                    pltpu.async_copy(
                        rows_v.at[pl.ds(j * 128, 128)],
                        out_hbm.at[dst_v.at[half * 4 + j]], ssem)
                    for j in range(ch // 128)
                ]
                for c in scatters:
                    c.wait()
            return carry

        lax.fori_loop(0, n_ch, body, 0)

    return k(table, src_idx, dst_idx)


def _tc_mlp(x_num, emb4, w1n, w1e, b1, w2, b2, w3, b3):
    b = x_num.shape[0]
    d_num = x_num.shape[1]
    blk = 1024
    grid = (b // blk,)

    def body(xn_ref, e_ref, w1n_ref, w1e_ref, b1_ref, w2_ref, b2_ref,
             w3_ref, b3_ref, out_ref):
        h = jnp.dot(xn_ref[...], w1n_ref[...],
                    preferred_element_type=jnp.float32)
        for t in range(N_FIELDS):
            xt = e_ref[:, t, :, :].reshape(blk, LANE)
            h += jnp.dot(xt, w1e_ref[t],
                         preferred_element_type=jnp.float32)
        h = jnp.maximum(h + b1_ref[...], 0.0)
        h = jnp.maximum(
            jnp.dot(h, w2_ref[...], preferred_element_type=jnp.float32)
            + b2_ref[...], 0.0)
        out_ref[...] = (
            jnp.dot(h, w3_ref[...], preferred_element_type=jnp.float32)
            + b3_ref[...])

    out = pl.pallas_call(
        body,
        grid=grid,
        in_specs=[
            pl.BlockSpec((blk, d_num), lambda i: (i, 0)),
            pl.BlockSpec((blk // 8, N_FIELDS, 8, LANE),
                         lambda i: (i, 0, 0, 0)),
            pl.BlockSpec((d_num, 128), lambda i: (0, 0)),
            pl.BlockSpec((N_FIELDS, LANE, 128), lambda i: (0, 0, 0)),
            pl.BlockSpec((1, 128), lambda i: (0, 0)),
            pl.BlockSpec((128, 64), lambda i: (0, 0)),
            pl.BlockSpec((1, 64), lambda i: (0, 0)),
            pl.BlockSpec((64, 1), lambda i: (0, 0)),
            pl.BlockSpec((1, 1), lambda i: (0, 0)),
        ],
        out_specs=pl.BlockSpec((blk, 1), lambda i: (i, 0)),
        out_shape=jax.ShapeDtypeStruct((b, 1), jnp.float32),
    )(x_num, emb4, w1n, w1e, b1.reshape(1, -1), w2, b2.reshape(1, -1),
      w3, b3.reshape(1, 1))
    return out[:, 0]


def kernel(x_num, x_cat, E, W1, b1, W2, b2, W3, b3):
    b = x_num.shape[0]
    card = E.shape[1]
    d_num = x_num.shape[1]

    Et = jnp.swapaxes(E, 1, 2)
    table = _tc_transpose_table(Et, card).reshape(N_FIELDS * card, LANE)

    offs = (jnp.arange(N_FIELDS, dtype=jnp.int32) * card)[None, :]
    src_idx = (x_cat + offs).reshape(b * N_FIELDS // 128, 128)
    bb = jnp.arange(b, dtype=jnp.int32)[:, None]
    ff = jnp.arange(N_FIELDS, dtype=jnp.int32)[None, :]
    dst = ((bb // 8) * N_FIELDS + ff) * 8 + (bb % 8)
    dst_idx = dst.reshape(b * N_FIELDS // 128, 128)

    emb = _sc_gather_scatter(table, src_idx, dst_idx)
    emb4 = emb.reshape(b // 8, N_FIELDS, 8, LANE)

    w1e = jnp.pad(W1[d_num:].reshape(N_FIELDS, EMB_DIM, 128),
                  ((0, 0), (0, LANE - EMB_DIM), (0, 0)))
    return _tc_mlp(x_num, emb4, W1[:d_num], w1e, b1, W2, b2, W3, b3)


# speedup vs baseline: 14.3309x; 1.0621x over previous
import functools

import jax
import jax.numpy as jnp
from jax import lax
from jax.experimental import pallas as pl
from jax.experimental.pallas import tpu as pltpu
from jax.experimental.pallas import tpu_sc as plsc

N_FIELDS = 26
EMB_DIM = 50
LANE = 128
VB = 16384


def _tc_transpose_table(Et, card):
    grid = (N_FIELDS, pl.cdiv(card, VB))

    def body(in_ref, out_ref):
        x = in_ref[0]
        xt = jnp.swapaxes(x, 0, 1)
        out_ref[0] = jnp.pad(xt, ((0, 0), (0, LANE - EMB_DIM)))

    return pl.pallas_call(
        body,
        grid=grid,
        in_specs=[pl.BlockSpec((1, EMB_DIM, VB), lambda f, v: (f, 0, v))],
        out_specs=pl.BlockSpec((1, VB, LANE), lambda f, v: (f, v, 0)),
        out_shape=jax.ShapeDtypeStruct((N_FIELDS, card, LANE), jnp.float32),
    )(Et)


def _sc_gather_scatter(table, src_idx, dst_idx):
    n = src_idx.shape[0] * src_idx.shape[1]
    info = plsc.get_sparse_core_info()
    nw = info.num_cores * info.num_subcores
    per_w = n // nw
    ch = 512
    pair = 1024
    n_ch = per_w // pair
    mesh = plsc.VectorSubcoreMesh(core_axis_name="c", subcore_axis_name="s")

    @functools.partial(
        pl.kernel,
        mesh=mesh,
        out_type=jax.ShapeDtypeStruct((n, LANE), jnp.float32),
        scratch_types=[
            pltpu.VMEM((8, 128), jnp.int32),
            pltpu.VMEM((8, 128), jnp.int32),
            pltpu.VMEM((256, LANE), jnp.float32),
            pltpu.VMEM((256, LANE), jnp.float32),
            pltpu.SemaphoreType.DMA,
            pltpu.SemaphoreType.DMA,
            pltpu.SemaphoreType.DMA,
            pltpu.SemaphoreType.DMA,
        ],
    )
    def k(table_hbm, src_hbm, dst_hbm, out_hbm, src_v, dst_v, rows_a,
          rows_b, gsem_a, gsem_b, ssem_a, ssem_b):
        wid = lax.axis_index("s") * info.num_cores + lax.axis_index("c")
        base = wid * per_w
        rows = (rows_a, rows_b)
        gsems = (gsem_a, gsem_b)
        ssems = (ssem_a, ssem_b)

        def body(i, carry):
            off = base + i * pair
            row0 = pl.multiple_of(off // 128, 8)
            pltpu.sync_copy(src_hbm.at[pl.ds(row0, 8)], src_v)
            pltpu.sync_copy(dst_hbm.at[pl.ds(row0, 8)], dst_v)
            pend = [None, None]
            for u in range(4):
                slot = u % 2
                if pend[slot] is not None:
                    for c in pend[slot]:
                        c.wait()
                gathers = [
                    pltpu.async_copy(
                        table_hbm.at[src_v.at[u * 2 + j]],
                        rows[slot].at[pl.ds(j * 128, 128)], gsems[slot])
                    for j in range(2)
                ]
                for c in gathers:
                    c.wait()
                pend[slot] = [
                    pltpu.async_copy(
                        rows[slot].at[pl.ds(j * 128, 128)],
                        out_hbm.at[dst_v.at[u * 2 + j]], ssems[slot])
                    for j in range(2)
                ]
            for slot in range(2):
                for c in pend[slot]:
                    c.wait()
            return carry

        lax.fori_loop(0, n_ch, body, 0)

    return k(table, src_idx, dst_idx)


def _tc_mlp(x_num, emb4, w1n, w1e, b1, w2, b2, w3, b3):
    b = x_num.shape[0]
    d_num = x_num.shape[1]
    blk = 1024
    grid = (b // blk,)

    def body(xn_ref, e_ref, w1n_ref, w1e_ref, b1_ref, w2_ref, b2_ref,
             w3_ref, b3_ref, out_ref):
        h = jnp.dot(xn_ref[...], w1n_ref[...],
                    preferred_element_type=jnp.float32)
        for t in range(N_FIELDS):
            xt = e_ref[:, t, :, :].reshape(blk, LANE)
            h += jnp.dot(xt, w1e_ref[t],
                         preferred_element_type=jnp.float32)
        h = jnp.maximum(h + b1_ref[...], 0.0)
        h = jnp.maximum(
            jnp.dot(h, w2_ref[...], preferred_element_type=jnp.float32)
            + b2_ref[...], 0.0)
        out_ref[...] = (
            jnp.dot(h, w3_ref[...], preferred_element_type=jnp.float32)
            + b3_ref[...])

    out = pl.pallas_call(
        body,
        grid=grid,
        in_specs=[
            pl.BlockSpec((blk, d_num), lambda i: (i, 0)),
            pl.BlockSpec((blk // 8, N_FIELDS, 8, LANE),
                         lambda i: (i, 0, 0, 0)),
            pl.BlockSpec((d_num, 128), lambda i: (0, 0)),
            pl.BlockSpec((N_FIELDS, LANE, 128), lambda i: (0, 0, 0)),
            pl.BlockSpec((1, 128), lambda i: (0, 0)),
            pl.BlockSpec((128, 64), lambda i: (0, 0)),
            pl.BlockSpec((1, 64), lambda i: (0, 0)),
            pl.BlockSpec((64, 1), lambda i: (0, 0)),
            pl.BlockSpec((1, 1), lambda i: (0, 0)),
        ],
        out_specs=pl.BlockSpec((blk, 1), lambda i: (i, 0)),
        out_shape=jax.ShapeDtypeStruct((b, 1), jnp.float32),
    )(x_num, emb4, w1n, w1e, b1.reshape(1, -1), w2, b2.reshape(1, -1),
      w3, b3.reshape(1, 1))
    return out[:, 0]


def kernel(x_num, x_cat, E, W1, b1, W2, b2, W3, b3):
    b = x_num.shape[0]
    card = E.shape[1]
    d_num = x_num.shape[1]

    Et = jnp.swapaxes(E, 1, 2)
    table = _tc_transpose_table(Et, card).reshape(N_FIELDS * card, LANE)

    offs = (jnp.arange(N_FIELDS, dtype=jnp.int32) * card)[None, :]
    src_idx = (x_cat + offs).reshape(b * N_FIELDS // 128, 128)
    bb = jnp.arange(b, dtype=jnp.int32)[:, None]
    ff = jnp.arange(N_FIELDS, dtype=jnp.int32)[None, :]
    dst = ((bb // 8) * N_FIELDS + ff) * 8 + (bb % 8)
    dst_idx = dst.reshape(b * N_FIELDS // 128, 128)

    emb = _sc_gather_scatter(table, src_idx, dst_idx)
    emb4 = emb.reshape(b // 8, N_FIELDS, 8, LANE)

    w1e = jnp.pad(W1[d_num:].reshape(N_FIELDS, EMB_DIM, 128),
                  ((0, 0), (0, LANE - EMB_DIM), (0, 0)))
    return _tc_mlp(x_num, emb4, W1[:d_num], w1e, b1, W2, b2, W3, b3)


# speedup vs baseline: 14.4622x; 1.0092x over previous
import functools

import jax
import jax.numpy as jnp
from jax import lax
from jax.experimental import pallas as pl
from jax.experimental.pallas import tpu as pltpu
from jax.experimental.pallas import tpu_sc as plsc

N_FIELDS = 26
NF_A = 16
NF_B = 10
EMB_DIM = 50
LANE = 128
VB = 16384


def _tc_transpose_table(Et, f0, nf, card):
    grid = (nf, pl.cdiv(card, VB))

    def body(in_ref, out_ref):
        x = in_ref[0]
        xt = jnp.swapaxes(x, 0, 1)
        out_ref[0] = jnp.pad(xt, ((0, 0), (0, LANE - EMB_DIM)))

    return pl.pallas_call(
        body,
        grid=grid,
        in_specs=[pl.BlockSpec((1, EMB_DIM, VB),
                               lambda f, v: (f + f0, 0, v))],
        out_specs=pl.BlockSpec((1, VB, LANE), lambda f, v: (f, v, 0)),
        out_shape=jax.ShapeDtypeStruct((nf, card, LANE), jnp.float32),
    )(Et)


def _sc_gather_scatter(table, src_idx, dst_idx):
    n = src_idx.shape[0] * src_idx.shape[1]
    info = plsc.get_sparse_core_info()
    nw = info.num_cores * info.num_subcores
    per_w = n // nw
    pair = 1024
    n_ch = per_w // pair
    mesh = plsc.VectorSubcoreMesh(core_axis_name="c", subcore_axis_name="s")

    @functools.partial(
        pl.kernel,
        mesh=mesh,
        out_type=jax.ShapeDtypeStruct((n, LANE), jnp.float32),
        scratch_types=[
            pltpu.VMEM((8, 128), jnp.int32),
            pltpu.VMEM((8, 128), jnp.int32),
            pltpu.VMEM((256, LANE), jnp.float32),
            pltpu.VMEM((256, LANE), jnp.float32),
            pltpu.SemaphoreType.DMA,
            pltpu.SemaphoreType.DMA,
            pltpu.SemaphoreType.DMA,
            pltpu.SemaphoreType.DMA,
        ],
    )
    def k(table_hbm, src_hbm, dst_hbm, out_hbm, src_v, dst_v, rows_a,
          rows_b, gsem_a, gsem_b, ssem_a, ssem_b):
        wid = lax.axis_index("s") * info.num_cores + lax.axis_index("c")
        base = wid * per_w
        rows = (rows_a, rows_b)
        gsems = (gsem_a, gsem_b)
        ssems = (ssem_a, ssem_b)

        def body(i, carry):
            off = base + i * pair
            row0 = pl.multiple_of(off // 128, 8)
            pltpu.sync_copy(src_hbm.at[pl.ds(row0, 8)], src_v)
            pltpu.sync_copy(dst_hbm.at[pl.ds(row0, 8)], dst_v)
            pend = [None, None]
            for u in range(4):
                slot = u % 2
                if pend[slot] is not None:
                    for c in pend[slot]:
                        c.wait()
                gathers = [
                    pltpu.async_copy(
                        table_hbm.at[src_v.at[u * 2 + j]],
                        rows[slot].at[pl.ds(j * 128, 128)], gsems[slot])
                    for j in range(2)
                ]
                for c in gathers:
                    c.wait()
                pend[slot] = [
                    pltpu.async_copy(
                        rows[slot].at[pl.ds(j * 128, 128)],
                        out_hbm.at[dst_v.at[u * 2 + j]], ssems[slot])
                    for j in range(2)
                ]
            for slot in range(2):
                for c in pend[slot]:
                    c.wait()
            return carry

        lax.fori_loop(0, n_ch, body, 0)

    return k(table, src_idx, dst_idx)


def _tc_mlp(x_num, emb4a, emb4b, w1n, w1e, b1, w2, b2, w3, b3):
    b = x_num.shape[0]
    d_num = x_num.shape[1]
    blk = 1024
    grid = (b // blk,)

    def body(xn_ref, ea_ref, eb_ref, w1n_ref, w1e_ref, b1_ref, w2_ref,
             b2_ref, w3_ref, b3_ref, out_ref):
        h = jnp.dot(xn_ref[...], w1n_ref[...],
                    preferred_element_type=jnp.float32)
        for t in range(N_FIELDS):
            if t < NF_A:
                xt = ea_ref[:, t, :, :].reshape(blk, LANE)
            else:
                xt = eb_ref[:, t - NF_A, :, :].reshape(blk, LANE)
            h += jnp.dot(xt, w1e_ref[t],
                         preferred_element_type=jnp.float32)
        h = jnp.maximum(h + b1_ref[...], 0.0)
        h = jnp.maximum(
            jnp.dot(h, w2_ref[...], preferred_element_type=jnp.float32)
            + b2_ref[...], 0.0)
        out_ref[...] = (
            jnp.dot(h, w3_ref[...], preferred_element_type=jnp.float32)
            + b3_ref[...])

    out = pl.pallas_call(
        body,
        grid=grid,
        in_specs=[
            pl.BlockSpec((blk, d_num), lambda i: (i, 0)),
            pl.BlockSpec((blk // 8, NF_A, 8, LANE), lambda i: (i, 0, 0, 0)),
            pl.BlockSpec((blk // 8, NF_B, 8, LANE), lambda i: (i, 0, 0, 0)),
            pl.BlockSpec((d_num, 128), lambda i: (0, 0)),
            pl.BlockSpec((N_FIELDS, LANE, 128), lambda i: (0, 0, 0)),
            pl.BlockSpec((1, 128), lambda i: (0, 0)),
            pl.BlockSpec((128, 64), lambda i: (0, 0)),
            pl.BlockSpec((1, 64), lambda i: (0, 0)),
            pl.BlockSpec((64, 1), lambda i: (0, 0)),
            pl.BlockSpec((1, 1), lambda i: (0, 0)),
        ],
        out_specs=pl.BlockSpec((blk, 1), lambda i: (i, 0)),
        out_shape=jax.ShapeDtypeStruct((b, 1), jnp.float32),
    )(x_num, emb4a, emb4b, w1n, w1e, b1.reshape(1, -1), w2,
      b2.reshape(1, -1), w3, b3.reshape(1, 1))
    return out[:, 0]


def _group(x_cat, b, card, f0, nf):
    offs = ((jnp.arange(nf, dtype=jnp.int32) + f0) * card)[None, :]
    src = (x_cat[:, f0:f0 + nf] + offs).reshape(b * nf // 128, 128)
    bb = jnp.arange(b, dtype=jnp.int32)[:, None]
    ff = jnp.arange(nf, dtype=jnp.int32)[None, :]
    dst = (((bb // 8) * nf + ff) * 8 + (bb % 8)).reshape(b * nf // 128, 128)
    return src, dst


def kernel(x_num, x_cat, E, W1, b1, W2, b2, W3, b3):
    b = x_num.shape[0]
    card = E.shape[1]
    d_num = x_num.shape[1]

    Et = jnp.swapaxes(E, 1, 2)
    tab_a = _tc_transpose_table(Et, 0, NF_A, card)
    src_a, dst_a = _group(x_cat, b, card, 0, NF_A)
    emb_a = _sc_gather_scatter(
        tab_a.reshape(NF_A * card, LANE), src_a, dst_a)
    tab_b = _tc_transpose_table(Et, NF_A, NF_B, card)
    src_b, dst_b = _group(x_cat, b, card, NF_A, NF_B)
    src_b = src_b - NF_A * card
    emb_b = _sc_gather_scatter(
        tab_b.reshape(NF_B * card, LANE), src_b, dst_b)

    emb4a = emb_a.reshape(b // 8, NF_A, 8, LANE)
    emb4b = emb_b.reshape(b // 8, NF_B, 8, LANE)

    w1e = jnp.pad(W1[d_num:].reshape(N_FIELDS, EMB_DIM, 128),
                  ((0, 0), (0, LANE - EMB_DIM), (0, 0)))
    return _tc_mlp(x_num, emb4a, emb4b, W1[:d_num], w1e, b1, W2, b2,
                   W3, b3)
